# baseline (speedup 1.0000x reference)
import functools

import jax
import jax.numpy as jnp
from jax import lax
from jax.experimental import pallas as pl
from jax.experimental.pallas import tpu as pltpu
from jax.experimental.pallas import tpu_sc as plsc

_B, _N, _F, _C = 8, 2048, 512, 32
_NSPLIT = 2
_BLK = _N // _NSPLIT
_NW = 32
_RPW = (_B * _C) // _NW


def _half_argmax(w, x, iota, base):
    scores_t = jax.lax.dot_general(
        w, x, (((1,), (1,)), ((), ())),
        preferred_element_type=jnp.float32,
    )
    hmax = jnp.max(scores_t, axis=1, keepdims=True)
    hidx = jnp.min(
        jnp.where(scores_t == hmax, iota, _BLK), axis=1, keepdims=True
    ) + base
    return hmax, hidx


def _score_kernel(xa_ref, xb_ref, w_ref, idx_ref):
    w = w_ref[...]
    iota = jax.lax.broadcasted_iota(jnp.int32, (_C, _BLK), 1)
    amax, aidx = _half_argmax(w, xa_ref[0], iota, 0)
    bmax, bidx = _half_argmax(w, xb_ref[0], iota, _BLK)
    better = bmax > amax
    idx_ref[0] = jnp.where(better, bidx, aidx) + pl.program_id(0) * _N


def _score(x, W):
    return pl.pallas_call(
        _score_kernel,
        grid=(_B,),
        in_specs=[
            pl.BlockSpec((1, _BLK, _F), lambda i: (i, 0, 0)),
            pl.BlockSpec((1, _BLK, _F), lambda i: (i, 1, 0)),
            pl.BlockSpec((_C, _F), lambda i: (0, 0)),
        ],
        out_specs=pl.BlockSpec((1, _C, 1), lambda i: (i, 0, 0)),
        out_shape=jax.ShapeDtypeStruct((_B, _C, 1), jnp.int32),
    )(x, x, W)


def _sc_gather_kernel(x_ref, i_ref, d_ref, idx_v, rows_v, sem):
    wid = lax.axis_index("s") * 2 + lax.axis_index("c")
    base = wid * _RPW
    pltpu.sync_copy(i_ref.at[pl.ds(base, _RPW)], idx_v)
    pltpu.async_copy(x_ref.at[idx_v], rows_v, sem).wait()
    pltpu.sync_copy(rows_v, d_ref.at[pl.ds(base, _RPW)])


@functools.cache
def _sc_gather_call():
    mesh = plsc.VectorSubcoreMesh(core_axis_name="c", subcore_axis_name="s")
    return pl.kernel(
        _sc_gather_kernel,
        out_type=jax.ShapeDtypeStruct((_B * _C, _F), jnp.float32),
        mesh=mesh,
        scratch_types=[
            pltpu.VMEM((_RPW,), jnp.int32),
            pltpu.VMEM((_RPW, _F), jnp.float32),
            pltpu.SemaphoreType.DMA,
        ],
    )


def kernel(x1, x2, W, b):
    del b
    gather = _sc_gather_call()
    idx1 = _score(x1, W)
    d = gather(x1.reshape(_B * _N, _F), idx1.reshape(_B * _C))
    idx2 = _score(x2, W)
    d1 = gather(x2.reshape(_B * _N, _F), idx2.reshape(_B * _C))
    return (d.reshape(_B, _C, _F), d1.reshape(_B, _C, _F))


# speedup vs baseline: 1.5795x; 1.5795x over previous
import jax
import jax.numpy as jnp
from jax.experimental import pallas as pl
from jax.experimental.pallas import tpu as pltpu

_B, _N, _F, _C = 8, 2048, 512, 32
_NSPLIT = 2
_BLK = _N // _NSPLIT


def _half_argmax(w, x, iota, base):
    scores_t = jax.lax.dot_general(
        w, x, (((1,), (1,)), ((), ())),
        preferred_element_type=jnp.float32,
    )
    hmax = jnp.max(scores_t, axis=1, keepdims=True)
    hidx = jnp.min(
        jnp.where(scores_t == hmax, iota, _BLK), axis=1, keepdims=True
    ) + base
    return hmax, hidx


def _score_kernel(x1a_ref, x1b_ref, x2a_ref, x2b_ref, w_ref,
                  idx1_ref, idx2_ref):
    w = w_ref[...]
    iota = jax.lax.broadcasted_iota(jnp.int32, (_C, _BLK), 1)
    for (a_ref, b_ref), idx_ref in (((x1a_ref, x1b_ref), idx1_ref),
                                    ((x2a_ref, x2b_ref), idx2_ref)):
        amax, aidx = _half_argmax(w, a_ref[0], iota, 0)
        bmax, bidx = _half_argmax(w, b_ref[0], iota, _BLK)
        better = bmax > amax
        idx_ref[0] = jnp.where(better, bidx, aidx)


def _gather_kernel(i1_ref, i2_ref, x1_ref, x2_ref, d_ref, d1_ref, sem):
    copies = []
    for b in range(_B):
        for c in range(_C):
            r1 = i1_ref[b, c, 0]
            r2 = i2_ref[b, c, 0]
            cp1 = pltpu.make_async_copy(
                x1_ref.at[b, pl.ds(r1, 1), :], d_ref.at[b, pl.ds(c, 1), :], sem)
            cp2 = pltpu.make_async_copy(
                x2_ref.at[b, pl.ds(r2, 1), :], d1_ref.at[b, pl.ds(c, 1), :], sem)
            cp1.start()
            cp2.start()
            copies.append(cp1)
            copies.append(cp2)
    for cp in copies:
        cp.wait()


def kernel(x1, x2, W, b):
    del b
    idx1, idx2 = pl.pallas_call(
        _score_kernel,
        grid=(_B,),
        in_specs=[
            pl.BlockSpec((1, _BLK, _F), lambda i: (i, 0, 0)),
            pl.BlockSpec((1, _BLK, _F), lambda i: (i, 1, 0)),
            pl.BlockSpec((1, _BLK, _F), lambda i: (i, 0, 0)),
            pl.BlockSpec((1, _BLK, _F), lambda i: (i, 1, 0)),
            pl.BlockSpec((_C, _F), lambda i: (0, 0)),
        ],
        out_specs=[
            pl.BlockSpec((1, _C, 1), lambda i: (i, 0, 0)),
            pl.BlockSpec((1, _C, 1), lambda i: (i, 0, 0)),
        ],
        out_shape=[
            jax.ShapeDtypeStruct((_B, _C, 1), jnp.int32),
            jax.ShapeDtypeStruct((_B, _C, 1), jnp.int32),
        ],
    )(x1, x1, x2, x2, W)

    d, d1 = pl.pallas_call(
        _gather_kernel,
        grid_spec=pltpu.PrefetchScalarGridSpec(
            num_scalar_prefetch=2,
            grid=(1,),
            in_specs=[
                pl.BlockSpec(memory_space=pl.ANY),
                pl.BlockSpec(memory_space=pl.ANY),
            ],
            out_specs=[
                pl.BlockSpec((_B, _C, _F), lambda i, i1, i2: (0, 0, 0)),
                pl.BlockSpec((_B, _C, _F), lambda i, i1, i2: (0, 0, 0)),
            ],
            scratch_shapes=[pltpu.SemaphoreType.DMA],
        ),
        out_shape=[
            jax.ShapeDtypeStruct((_B, _C, _F), jnp.float32),
            jax.ShapeDtypeStruct((_B, _C, _F), jnp.float32),
        ],
    )(idx1, idx2, x1, x2)
    return (d, d1)
